# baseline (speedup 1.0000x reference)
import jax
import jax.numpy as jnp
from jax.experimental import pallas as pl
from jax.experimental.pallas import tpu as pltpu

N, D = 100000, 128
BLOCK = 2000


def _emma_body(x_ref, max_a_ref, agg_n_ref, his_x_ref, his_m_ref, inv_w_ref,
               out_ref):
    max_a = max_a_ref[...]
    his_m = his_m_ref[...]
    beta = jnp.clip(1.0 - inv_w_ref[...] * agg_n_ref[...], 0.0, 1.0)
    max_m = jnp.maximum(max_a, his_m)
    neg_inf = jnp.float32(-jnp.inf)
    dp = his_m - max_m
    dq = max_a - max_m
    dp = jnp.where(jnp.isnan(dp), neg_inf, dp)
    dq = jnp.where(jnp.isnan(dq), neg_inf, dq)
    p = jnp.exp(dp) * beta
    q = jnp.exp(dq)
    t = jnp.maximum(p + q, 1.0)
    inv_t = 1.0 / t
    p = p * inv_t
    q = q * inv_t
    out_ref[...] = his_x_ref[...] * p + x_ref[...] * q


def kernel(x, max_a, agg_n, his_x, his_m, inv_w):
    n = x.shape[0]
    max_a2 = max_a.reshape(n, 1)
    agg_n2 = agg_n.reshape(n, 1)
    his_m2 = his_m.reshape(n, 1)
    inv_w2 = inv_w.reshape(n, 1)
    grid = n // BLOCK
    row_spec = pl.BlockSpec((BLOCK, D), lambda i: (i, 0))
    vec_spec = pl.BlockSpec((BLOCK, 1), lambda i: (i, 0))
    return pl.pallas_call(
        _emma_body,
        grid=(grid,),
        in_specs=[row_spec, vec_spec, vec_spec, row_spec, vec_spec, vec_spec],
        out_specs=row_spec,
        out_shape=jax.ShapeDtypeStruct((n, D), jnp.float32),
        compiler_params=pltpu.CompilerParams(
            dimension_semantics=("arbitrary",),
        ),
    )(x, max_a2, agg_n2, his_x, his_m2, inv_w2)


# speedup vs baseline: 3.9677x; 3.9677x over previous
import jax
import jax.numpy as jnp
from jax.experimental import pallas as pl
from jax.experimental.pallas import tpu as pltpu

N, D = 100000, 128
BLOCK = 2048


def _emma_body(x_ref, max_a_ref, agg_n_ref, his_x_ref, his_m_ref, inv_w_ref,
               out_ref):
    max_a = max_a_ref[...]
    his_m = his_m_ref[...]
    beta = jnp.clip(1.0 - inv_w_ref[...] * agg_n_ref[...], 0.0, 1.0)
    max_m = jnp.maximum(max_a, his_m)
    neg_inf = jnp.float32(-jnp.inf)
    dp = his_m - max_m
    dq = max_a - max_m
    dp = jnp.where(jnp.isnan(dp), neg_inf, dp)
    dq = jnp.where(jnp.isnan(dq), neg_inf, dq)
    p = jnp.exp(dp) * beta
    q = jnp.exp(dq)
    t = jnp.maximum(p + q, 1.0)
    inv_t = 1.0 / t
    p2 = (p * inv_t)[:, None]
    q2 = (q * inv_t)[:, None]
    out_ref[...] = his_x_ref[...] * p2 + x_ref[...] * q2


def kernel(x, max_a, agg_n, his_x, his_m, inv_w):
    n = x.shape[0]
    grid = (n + BLOCK - 1) // BLOCK
    row_spec = pl.BlockSpec((BLOCK, D), lambda i: (i, 0))
    vec_spec = pl.BlockSpec((BLOCK,), lambda i: (i,))
    return pl.pallas_call(
        _emma_body,
        grid=(grid,),
        in_specs=[row_spec, vec_spec, vec_spec, row_spec, vec_spec, vec_spec],
        out_specs=row_spec,
        out_shape=jax.ShapeDtypeStruct((n, D), jnp.float32),
        compiler_params=pltpu.CompilerParams(
            dimension_semantics=("arbitrary",),
        ),
    )(x, max_a, agg_n, his_x, his_m, inv_w)


# speedup vs baseline: 4.9677x; 1.2520x over previous
import jax
import jax.numpy as jnp
from jax.experimental import pallas as pl
from jax.experimental.pallas import tpu as pltpu

N, D = 100000, 128
BLOCK = 4096


def _emma_body(x_ref, max_a_ref, agg_n_ref, his_x_ref, his_m_ref, inv_w_ref,
               out_ref):
    max_a = max_a_ref[...]
    his_m = his_m_ref[...]
    beta = jnp.clip(1.0 - inv_w_ref[...] * agg_n_ref[...], 0.0, 1.0)
    max_m = jnp.maximum(max_a, his_m)
    neg_inf = jnp.float32(-jnp.inf)
    dp = his_m - max_m
    dq = max_a - max_m
    dp = jnp.where(jnp.isnan(dp), neg_inf, dp)
    dq = jnp.where(jnp.isnan(dq), neg_inf, dq)
    p = jnp.exp(dp) * beta
    q = jnp.exp(dq)
    t = jnp.maximum(p + q, 1.0)
    inv_t = 1.0 / t
    p2 = (p * inv_t)[:, None]
    q2 = (q * inv_t)[:, None]
    out_ref[...] = his_x_ref[...] * p2 + x_ref[...] * q2


def kernel(x, max_a, agg_n, his_x, his_m, inv_w):
    n = x.shape[0]
    grid = (n + BLOCK - 1) // BLOCK
    row_spec = pl.BlockSpec((BLOCK, D), lambda i: (i, 0))
    vec_spec = pl.BlockSpec((BLOCK,), lambda i: (i,))
    return pl.pallas_call(
        _emma_body,
        grid=(grid,),
        in_specs=[row_spec, vec_spec, vec_spec, row_spec, vec_spec, vec_spec],
        out_specs=row_spec,
        out_shape=jax.ShapeDtypeStruct((n, D), jnp.float32),
        compiler_params=pltpu.CompilerParams(
            dimension_semantics=("arbitrary",),
        ),
    )(x, max_a, agg_n, his_x, his_m, inv_w)


# speedup vs baseline: 5.1611x; 1.0389x over previous
import jax
import jax.numpy as jnp
from jax.experimental import pallas as pl
from jax.experimental.pallas import tpu as pltpu

N, D = 100000, 128
BLOCK = 8192


def _emma_body(x_ref, max_a_ref, agg_n_ref, his_x_ref, his_m_ref, inv_w_ref,
               out_ref):
    max_a = max_a_ref[...]
    his_m = his_m_ref[...]
    beta = jnp.clip(1.0 - inv_w_ref[...] * agg_n_ref[...], 0.0, 1.0)
    max_m = jnp.maximum(max_a, his_m)
    neg_inf = jnp.float32(-jnp.inf)
    dp = his_m - max_m
    dq = max_a - max_m
    dp = jnp.where(jnp.isnan(dp), neg_inf, dp)
    dq = jnp.where(jnp.isnan(dq), neg_inf, dq)
    p = jnp.exp(dp) * beta
    q = jnp.exp(dq)
    t = jnp.maximum(p + q, 1.0)
    inv_t = 1.0 / t
    p2 = (p * inv_t)[:, None]
    q2 = (q * inv_t)[:, None]
    out_ref[...] = his_x_ref[...] * p2 + x_ref[...] * q2


def kernel(x, max_a, agg_n, his_x, his_m, inv_w):
    n = x.shape[0]
    grid = (n + BLOCK - 1) // BLOCK
    row_spec = pl.BlockSpec((BLOCK, D), lambda i: (i, 0))
    vec_spec = pl.BlockSpec((BLOCK,), lambda i: (i,))
    return pl.pallas_call(
        _emma_body,
        grid=(grid,),
        in_specs=[row_spec, vec_spec, vec_spec, row_spec, vec_spec, vec_spec],
        out_specs=row_spec,
        out_shape=jax.ShapeDtypeStruct((n, D), jnp.float32),
        compiler_params=pltpu.CompilerParams(
            dimension_semantics=("arbitrary",),
        ),
    )(x, max_a, agg_n, his_x, his_m, inv_w)


# speedup vs baseline: 5.3309x; 1.0329x over previous
import jax
import jax.numpy as jnp
from jax.experimental import pallas as pl
from jax.experimental.pallas import tpu as pltpu

N, D = 100000, 128
BLOCK = 16384


def _emma_body(x_ref, max_a_ref, agg_n_ref, his_x_ref, his_m_ref, inv_w_ref,
               out_ref):
    max_a = max_a_ref[...]
    his_m = his_m_ref[...]
    beta = jnp.clip(1.0 - inv_w_ref[...] * agg_n_ref[...], 0.0, 1.0)
    max_m = jnp.maximum(max_a, his_m)
    neg_inf = jnp.float32(-jnp.inf)
    dp = his_m - max_m
    dq = max_a - max_m
    dp = jnp.where(jnp.isnan(dp), neg_inf, dp)
    dq = jnp.where(jnp.isnan(dq), neg_inf, dq)
    p = jnp.exp(dp) * beta
    q = jnp.exp(dq)
    t = jnp.maximum(p + q, 1.0)
    inv_t = 1.0 / t
    p2 = (p * inv_t)[:, None]
    q2 = (q * inv_t)[:, None]
    out_ref[...] = his_x_ref[...] * p2 + x_ref[...] * q2


def kernel(x, max_a, agg_n, his_x, his_m, inv_w):
    n = x.shape[0]
    grid = (n + BLOCK - 1) // BLOCK
    row_spec = pl.BlockSpec((BLOCK, D), lambda i: (i, 0))
    vec_spec = pl.BlockSpec((BLOCK,), lambda i: (i,))
    return pl.pallas_call(
        _emma_body,
        grid=(grid,),
        in_specs=[row_spec, vec_spec, vec_spec, row_spec, vec_spec, vec_spec],
        out_specs=row_spec,
        out_shape=jax.ShapeDtypeStruct((n, D), jnp.float32),
        compiler_params=pltpu.CompilerParams(
            dimension_semantics=("arbitrary",),
        ),
    )(x, max_a, agg_n, his_x, his_m, inv_w)
